# baseline (speedup 1.0000x reference)
import jax
import jax.numpy as jnp
from jax import lax
from jax.experimental import pallas as pl
from jax.experimental.pallas import tpu as pltpu
from jax.experimental.pallas import tpu_sc as plsc

N_NODES = 10000
N_EDGES = 320000
D = 128
ED = 16

NUM_CORES = 2
NUM_SUBCORES = 16
TILES = NUM_CORES * NUM_SUBCORES
EDGES_PER_TILE = N_EDGES // TILES
BLK_A = 40
NBLK_A = EDGES_PER_TILE // BLK_A
ROWS_MAIN = 624
ROWS_TAIL = N_NODES - ROWS_MAIN * NUM_SUBCORES


def _node_mm_body(x_ref, w_ref, b_ref, ux_ref, vxb_ref, a_ref):
    full = jnp.dot(x_ref[...], w_ref[...], preferred_element_type=jnp.float32)
    full = full + b_ref[...]
    ux_ref[...] = full[:, :D]
    vxb_ref[...] = full[:, D:3 * D]
    a_ref[...] = full[:, 3 * D:]


def _node_matmuls(x, w_cat, b_cat):
    return pl.pallas_call(
        _node_mm_body,
        out_shape=(
            jax.ShapeDtypeStruct((N_NODES, D), jnp.float32),
            jax.ShapeDtypeStruct((N_NODES, 2 * D), jnp.float32),
            jax.ShapeDtypeStruct((N_NODES, D), jnp.float32),
        ),
    )(x, w_cat, b_cat)


E_BLK = 8000


def _e_body(ea_ref, w_ref, b_ref, o_ref):
    o_ref[...] = jax.nn.sigmoid(
        jnp.dot(ea_ref[...], w_ref[...], preferred_element_type=jnp.float32)
        + b_ref[...])


def _edge_gate(edge_attr, e_wt, e_b):
    return pl.pallas_call(
        _e_body,
        grid=(N_EDGES // E_BLK,),
        in_specs=[
            pl.BlockSpec((E_BLK, ED), lambda i: (i, 0)),
            pl.BlockSpec((ED, D), lambda i: (0, 0)),
            pl.BlockSpec((1, D), lambda i: (0, 0)),
        ],
        out_specs=pl.BlockSpec((E_BLK, D), lambda i: (i, 0)),
        out_shape=jax.ShapeDtypeStruct((N_EDGES, D), jnp.float32),
    )(edge_attr, e_wt, e_b)


def _sc_edge_pass(vxb_tab, a_tab, e, src, dst, zrows):
    mesh = plsc.VectorSubcoreMesh(core_axis_name="core",
                                  subcore_axis_name="subcore")

    @pl.kernel(
        out_type=(
            jax.ShapeDtypeStruct((N_EDGES, 32), jnp.float32),
            jax.ShapeDtypeStruct((NUM_CORES * N_NODES, D), jnp.float32),
        ),
        mesh=mesh,
        scratch_types=[
            pltpu.VMEM((BLK_A,), jnp.int32),
            pltpu.VMEM((BLK_A,), jnp.int32),
            pltpu.VMEM((BLK_A,), jnp.int32),
            pltpu.VMEM((BLK_A,), jnp.int32),
            pltpu.VMEM((BLK_A, 2 * D), jnp.float32),
            pltpu.VMEM((BLK_A, 2 * D), jnp.float32),
            pltpu.VMEM((BLK_A, D), jnp.float32),
            pltpu.VMEM((BLK_A, D), jnp.float32),
            pltpu.VMEM((BLK_A, D), jnp.float32),
            pltpu.VMEM((BLK_A, D), jnp.float32),
            pltpu.VMEM((BLK_A, 32), jnp.float32),
            pltpu.VMEM_SHARED((N_NODES, D), jnp.float32),
            pltpu.SemaphoreType.DMA,
            pltpu.SemaphoreType.DMA,
        ],
    )
    def sc_kernel(vxb_hbm, a_hbm, e_hbm, src_hbm, dst_hbm, z_hbm,
                  s_hbm, agg_hbm,
                  is0, is1, id0, id1, r0, r1, ev0, ev1, ga0, ga1, sv,
                  agg_sh, sem_i, sem_g):
        cid = lax.axis_index("core")
        sid = lax.axis_index("subcore")

        idx_s = (is0, is1)
        idx_d = (id0, id1)
        rows = (r0, r1)
        ev = (ev0, ev1)
        ga = (ga0, ga1)

        pltpu.sync_copy(z_hbm.at[pl.ds(0, ROWS_MAIN)],
                        agg_sh.at[pl.ds(sid * ROWS_MAIN, ROWS_MAIN)])

        @pl.when(sid == NUM_SUBCORES - 1)
        def _():
            pltpu.sync_copy(
                z_hbm.at[pl.ds(0, ROWS_TAIL)],
                agg_sh.at[pl.ds(NUM_SUBCORES * ROWS_MAIN, ROWS_TAIL)])

        plsc.subcore_barrier()

        base0 = (cid * NUM_SUBCORES + sid) * EDGES_PER_TILE

        def issue_idx(b, p):
            base = base0 + b * BLK_A
            pltpu.async_copy(src_hbm.at[pl.ds(base, BLK_A)], idx_s[p], sem_i)
            pltpu.async_copy(dst_hbm.at[pl.ds(base, BLK_A)], idx_d[p], sem_i)

        def wait_idx(p):
            pltpu.make_async_copy(
                src_hbm.at[pl.ds(0, BLK_A)], idx_s[p], sem_i).wait()
            pltpu.make_async_copy(
                dst_hbm.at[pl.ds(0, BLK_A)], idx_d[p], sem_i).wait()

        def issue_gathers(b, p):
            base = base0 + b * BLK_A
            pltpu.async_copy(vxb_hbm.at[idx_d[p]], rows[p], sem_g)
            pltpu.async_copy(a_hbm.at[idx_s[p]], ga[p], sem_g)
            pltpu.async_copy(e_hbm.at[pl.ds(base, BLK_A)], ev[p], sem_g)

        def wait_gathers(p):
            pltpu.make_async_copy(
                vxb_hbm.at[pl.ds(0, BLK_A)], rows[p], sem_g).wait()
            pltpu.make_async_copy(
                e_hbm.at[pl.ds(0, BLK_A)], ga[p], sem_g).wait()
            pltpu.make_async_copy(
                e_hbm.at[pl.ds(0, BLK_A)], ev[p], sem_g).wait()

        def phase(b, p, q):
            @pl.when(b + 1 < NBLK_A)
            def _():
                wait_idx(q)
                issue_gathers(b + 1, q)

            wait_gathers(p)
            evp, rowsp, gap = ev[p], rows[p], ga[p]

            @pl.loop(0, BLK_A)
            def _(i):
                r = pl.ds(i, 1)
                evp.at[r, :][...] = (evp.at[r, :][...]
                                     * rowsp.at[r, pl.ds(0, D)][...])

            sv[...] = gap[:, :32] + rowsp[:, D:D + 32]

            pltpu.sync_copy(ev[p], agg_sh.at[idx_s[p]], add=True)
            base = base0 + b * BLK_A
            pltpu.sync_copy(sv, s_hbm.at[pl.ds(base, BLK_A)])

            @pl.when(b + 2 < NBLK_A)
            def _():
                issue_idx(b + 2, p)

        issue_idx(0, 0)
        issue_idx(1, 1)
        wait_idx(0)
        issue_gathers(0, 0)

        @pl.loop(0, NBLK_A, step=2)
        def _(t):
            phase(t, 0, 1)

            @pl.when(t + 1 < NBLK_A)
            def _():
                phase(t + 1, 1, 0)

        plsc.subcore_barrier()
        pltpu.sync_copy(
            agg_sh.at[pl.ds(sid * ROWS_MAIN, ROWS_MAIN)],
            agg_hbm.at[pl.ds(cid * N_NODES + sid * ROWS_MAIN, ROWS_MAIN)])

        @pl.when(sid == NUM_SUBCORES - 1)
        def _():
            pltpu.sync_copy(
                agg_sh.at[pl.ds(NUM_SUBCORES * ROWS_MAIN, ROWS_TAIL)],
                agg_hbm.at[pl.ds(cid * N_NODES + NUM_SUBCORES * ROWS_MAIN,
                                 ROWS_TAIL)])

    return sc_kernel(vxb_tab, a_tab, e, src, dst, zrows)


def _node_update_body(x_ref, ux_ref, agg_ref, gwu_ref, gwa_ref, gb_ref,
                      gamma_ref, beta_ref, o_ref):
    ux = ux_ref[...]
    agg = agg_ref[:N_NODES, :] + agg_ref[N_NODES:, :]
    logit = jnp.sum(ux * gwu_ref[...] + agg * gwa_ref[...],
                    axis=1, keepdims=True) + gb_ref[0:1, 0:1]
    gate = jax.nn.sigmoid(logit)
    h = gate * ux + (1.0 - gate) * agg
    mean = jnp.mean(h, axis=0, keepdims=True)
    var = jnp.mean((h - mean) ** 2, axis=0, keepdims=True)
    h_norm = (h - mean) / jnp.sqrt(var + 1e-5) * gamma_ref[...] + beta_ref[...]
    o_ref[...] = x_ref[...] + jax.nn.relu(h_norm)


def _node_update(x, ux, agg_partials, gwu, gwa, gb, gamma, beta):
    return pl.pallas_call(
        _node_update_body,
        out_shape=jax.ShapeDtypeStruct((N_NODES, D), jnp.float32),
    )(x, ux, agg_partials, gwu, gwa, gb, gamma, beta)


def _edge_mlp_body(ea_ref, s_ref, w3t_ref, b1_ref, w2t_ref, b2_ref, o_ref):
    c = jnp.dot(ea_ref[...], w3t_ref[...], preferred_element_type=jnp.float32)
    z = jax.nn.relu(s_ref[...] + c + b1_ref[...])
    o_ref[...] = jnp.dot(z, w2t_ref[...],
                         preferred_element_type=jnp.float32) + b2_ref[...]


def _edge_mlp(edge_attr, s, w3t, b1, w2t, b2):
    return pl.pallas_call(
        _edge_mlp_body,
        grid=(N_EDGES // E_BLK,),
        in_specs=[
            pl.BlockSpec((E_BLK, ED), lambda i: (i, 0)),
            pl.BlockSpec((E_BLK, 32), lambda i: (i, 0)),
            pl.BlockSpec((ED, 32), lambda i: (0, 0)),
            pl.BlockSpec((1, 32), lambda i: (0, 0)),
            pl.BlockSpec((32, ED), lambda i: (0, 0)),
            pl.BlockSpec((1, ED), lambda i: (0, 0)),
        ],
        out_specs=pl.BlockSpec((E_BLK, ED), lambda i: (i, 0)),
        out_shape=jax.ShapeDtypeStruct((N_EDGES, ED), jnp.float32),
    )(edge_attr, s, w3t, b1, w2t, b2)


def kernel(x, edge_index, edge_attr, U_w, U_b, V_w, V_b, E_w, E_b,
           gate_w, gate_b, eu1_w, eu1_b, eu2_w, eu2_b, bn_gamma, bn_beta):
    src = edge_index[0].astype(jnp.int32)
    dst = edge_index[1].astype(jnp.int32)

    w_cat = jnp.concatenate(
        [U_w.T, V_w.T, eu1_w[:, D:2 * D].T, jnp.zeros((D, 96), jnp.float32),
         eu1_w[:, :D].T, jnp.zeros((D, 96), jnp.float32)], axis=1)
    b_cat = jnp.concatenate(
        [U_b, V_b, jnp.zeros((2 * D,), jnp.float32)])[None, :]

    ux, vxb_tab, a_tab = _node_matmuls(x, w_cat, b_cat)
    e = _edge_gate(edge_attr, E_w.T, E_b[None, :])

    zrows = jnp.zeros((ROWS_MAIN, D), jnp.float32)
    s_sum, agg_partials = _sc_edge_pass(vxb_tab, a_tab, e, src, dst, zrows)

    gwu = gate_w[:, :D]
    gwa = gate_w[:, D:]
    gb = jnp.broadcast_to(gate_b.reshape(1, 1), (1, D))
    x_new = _node_update(x, ux, agg_partials, gwu, gwa, gb,
                         bn_gamma[None, :], bn_beta[None, :])

    upd = _edge_mlp(edge_attr, s_sum, eu1_w[:, 2 * D:].T, eu1_b[None, :],
                    eu2_w.T, eu2_b[None, :])
    return x_new, upd


# speedup vs baseline: 1.3243x; 1.3243x over previous
import jax
import jax.numpy as jnp
from jax import lax
from jax.experimental import pallas as pl
from jax.experimental.pallas import tpu as pltpu
from jax.experimental.pallas import tpu_sc as plsc

N_NODES = 10000
N_EDGES = 320000
D = 128
ED = 16

NUM_CORES = 2
NUM_SUBCORES = 16
TILES = NUM_CORES * NUM_SUBCORES
EDGES_PER_TILE = N_EDGES // TILES
BLK_A = 80
NBLK_A = EDGES_PER_TILE // BLK_A
BLK_B = 80
NBLK_B = EDGES_PER_TILE // BLK_B
ROWS_MAIN = 624
ROWS_TAIL = N_NODES - ROWS_MAIN * NUM_SUBCORES


def _node_mm_body(x_ref, w_ref, b_ref, ux_ref, vx_ref, ab_ref):
    full = jnp.dot(x_ref[...], w_ref[...], preferred_element_type=jnp.float32)
    full = full + b_ref[...]
    ux_ref[...] = full[:, :D]
    vx_ref[...] = full[:, D:2 * D]
    ab_ref[...] = full[:, 2 * D:]


def _node_matmuls(x, w_cat, b_cat):
    return pl.pallas_call(
        _node_mm_body,
        out_shape=(
            jax.ShapeDtypeStruct((N_NODES, D), jnp.float32),
            jax.ShapeDtypeStruct((N_NODES, D), jnp.float32),
            jax.ShapeDtypeStruct((N_NODES, D), jnp.float32),
        ),
    )(x, w_cat, b_cat)


E_BLK = 8000


def _e_body(ea_ref, w_ref, b_ref, o_ref):
    o_ref[...] = jax.nn.sigmoid(
        jnp.dot(ea_ref[...], w_ref[...], preferred_element_type=jnp.float32)
        + b_ref[...]).astype(jnp.bfloat16)


def _edge_gate(edge_attr, e_wt, e_b):
    return pl.pallas_call(
        _e_body,
        grid=(N_EDGES // E_BLK,),
        in_specs=[
            pl.BlockSpec((E_BLK, ED), lambda i: (i, 0)),
            pl.BlockSpec((ED, D), lambda i: (0, 0)),
            pl.BlockSpec((1, D), lambda i: (0, 0)),
        ],
        out_specs=pl.BlockSpec((E_BLK, D), lambda i: (i, 0)),
        out_shape=jax.ShapeDtypeStruct((N_EDGES, D), jnp.bfloat16),
    )(edge_attr, e_wt, e_b)


def _sc_messages(vx_tab, e, src, dst, zrows):
    mesh = plsc.VectorSubcoreMesh(core_axis_name="core",
                                  subcore_axis_name="subcore")

    @pl.kernel(
        out_type=jax.ShapeDtypeStruct((NUM_CORES * N_NODES, D), jnp.float32),
        mesh=mesh,
        scratch_types=[
            pltpu.VMEM((BLK_A,), jnp.int32),
            pltpu.VMEM((BLK_A,), jnp.int32),
            pltpu.VMEM((BLK_A,), jnp.int32),
            pltpu.VMEM((BLK_A,), jnp.int32),
            pltpu.VMEM((BLK_A, D), jnp.float32),
            pltpu.VMEM((BLK_A, D), jnp.float32),
            pltpu.VMEM((BLK_A, D), jnp.bfloat16),
            pltpu.VMEM((BLK_A, D), jnp.bfloat16),
            pltpu.VMEM((BLK_A,), jnp.int32),
            pltpu.VMEM((BLK_A,), jnp.int32),
            pltpu.VMEM_SHARED((N_NODES, D), jnp.float32),
            pltpu.SemaphoreType.DMA,
            pltpu.SemaphoreType.DMA,
            pltpu.SemaphoreType.DMA,
        ],
    )
    def sc_kernel(vx_hbm, e_hbm, src_hbm, dst_hbm, z_hbm, agg_hbm,
                  is0, is1, id0, id1, r0, r1, ev0, ev1, ss0, ss1, agg_sh,
                  sem_i, sem_g, sem_w):
        cid = lax.axis_index("core")
        sid = lax.axis_index("subcore")

        idx_s = (is0, is1)
        idx_d = (id0, id1)
        rows = (r0, r1)
        ev = (ev0, ev1)
        sidx = (ss0, ss1)

        pltpu.sync_copy(z_hbm.at[pl.ds(0, ROWS_MAIN)],
                        agg_sh.at[pl.ds(sid * ROWS_MAIN, ROWS_MAIN)])

        @pl.when(sid == NUM_SUBCORES - 1)
        def _():
            pltpu.sync_copy(
                z_hbm.at[pl.ds(0, ROWS_TAIL)],
                agg_sh.at[pl.ds(NUM_SUBCORES * ROWS_MAIN, ROWS_TAIL)])

        plsc.subcore_barrier()

        base0 = (cid * NUM_SUBCORES + sid) * EDGES_PER_TILE

        def issue_idx(b, p):
            base = base0 + b * BLK_A
            pltpu.async_copy(src_hbm.at[pl.ds(base, BLK_A)], idx_s[p], sem_i)
            pltpu.async_copy(dst_hbm.at[pl.ds(base, BLK_A)], idx_d[p], sem_i)

        def wait_idx(p):
            pltpu.make_async_copy(
                src_hbm.at[pl.ds(0, BLK_A)], idx_s[p], sem_i).wait()
            pltpu.make_async_copy(
                dst_hbm.at[pl.ds(0, BLK_A)], idx_d[p], sem_i).wait()

        def issue_gathers(b, p):
            base = base0 + b * BLK_A
            pltpu.async_copy(vx_hbm.at[idx_d[p]], rows[p], sem_g)
            pltpu.async_copy(e_hbm.at[pl.ds(base, BLK_A)], ev[p], sem_g)

        def wait_gathers(p):
            pltpu.make_async_copy(
                vx_hbm.at[pl.ds(0, BLK_A)], rows[p], sem_g).wait()
            pltpu.make_async_copy(
                e_hbm.at[pl.ds(0, BLK_A)], ev[p], sem_g).wait()

        def wait_scatter(p):
            pltpu.make_async_copy(
                vx_hbm.at[pl.ds(0, BLK_A)], rows[p], sem_w).wait()

        def phase(b, p, q):
            @pl.when(b + 1 < NBLK_A)
            def _():
                wait_idx(q)

                @pl.when(b >= 1)
                def _():
                    wait_scatter(q)

                issue_gathers(b + 1, q)

            wait_gathers(p)
            evp, rowsp = ev[p], rows[p]

            @pl.loop(0, BLK_A, step=2)
            def _(i):
                r = pl.ds(i, 2)
                rowsp.at[r, :][...] = (rowsp.at[r, :][...]
                                       * evp.at[r, :][...].astype(jnp.float32))

            sidx[p][...] = idx_s[p][...]
            pltpu.async_copy(rows[p], agg_sh.at[sidx[p]], sem_w, add=True)

            @pl.when(b + 2 < NBLK_A)
            def _():
                issue_idx(b + 2, p)

        issue_idx(0, 0)
        issue_idx(1, 1)
        wait_idx(0)
        issue_gathers(0, 0)

        @pl.loop(0, NBLK_A, step=2)
        def _(t):
            phase(t, 0, 1)

            @pl.when(t + 1 < NBLK_A)
            def _():
                phase(t + 1, 1, 0)

        wait_scatter(0)
        wait_scatter(1)

        plsc.subcore_barrier()
        pltpu.sync_copy(
            agg_sh.at[pl.ds(sid * ROWS_MAIN, ROWS_MAIN)],
            agg_hbm.at[pl.ds(cid * N_NODES + sid * ROWS_MAIN, ROWS_MAIN)])

        @pl.when(sid == NUM_SUBCORES - 1)
        def _():
            pltpu.sync_copy(
                agg_sh.at[pl.ds(NUM_SUBCORES * ROWS_MAIN, ROWS_TAIL)],
                agg_hbm.at[pl.ds(cid * N_NODES + NUM_SUBCORES * ROWS_MAIN,
                                 ROWS_TAIL)])

    return sc_kernel(vx_tab, e, src, dst, zrows)


def _sc_edge_sums(ab_tab, src, dst):
    mesh = plsc.VectorSubcoreMesh(core_axis_name="core",
                                  subcore_axis_name="subcore")

    @pl.kernel(
        out_type=jax.ShapeDtypeStruct((N_EDGES, 32), jnp.float32),
        mesh=mesh,
        scratch_types=[
            pltpu.VMEM((BLK_B,), jnp.int32),
            pltpu.VMEM((BLK_B,), jnp.int32),
            pltpu.VMEM((BLK_B,), jnp.int32),
            pltpu.VMEM((BLK_B,), jnp.int32),
            pltpu.VMEM((BLK_B, D), jnp.float32),
            pltpu.VMEM((BLK_B, D), jnp.float32),
            pltpu.VMEM((BLK_B, D), jnp.float32),
            pltpu.VMEM((BLK_B, D), jnp.float32),
            pltpu.VMEM((BLK_B, 32), jnp.float32),
            pltpu.VMEM((BLK_B, 32), jnp.float32),
            pltpu.SemaphoreType.DMA,
            pltpu.SemaphoreType.DMA,
            pltpu.SemaphoreType.DMA,
        ],
    )
    def sc_kernel(ab_hbm, src_hbm, dst_hbm, s_hbm,
                  is0, is1, id0, id1, gs0, gs1, gd0, gd1, sv0, sv1,
                  sem_i, sem_g, sem_w):
        cid = lax.axis_index("core")
        sid = lax.axis_index("subcore")

        idx_s = (is0, is1)
        idx_d = (id0, id1)
        gs = (gs0, gs1)
        gd = (gd0, gd1)
        sv = (sv0, sv1)

        base0 = (cid * NUM_SUBCORES + sid) * EDGES_PER_TILE

        def issue_idx(b, p):
            base = base0 + b * BLK_B
            pltpu.async_copy(src_hbm.at[pl.ds(base, BLK_B)], idx_s[p], sem_i)
            pltpu.async_copy(dst_hbm.at[pl.ds(base, BLK_B)], idx_d[p], sem_i)

        def wait_idx(p):
            pltpu.make_async_copy(
                src_hbm.at[pl.ds(0, BLK_B)], idx_s[p], sem_i).wait()
            pltpu.make_async_copy(
                dst_hbm.at[pl.ds(0, BLK_B)], idx_d[p], sem_i).wait()

        def issue_gathers(p):
            pltpu.async_copy(ab_hbm.at[idx_s[p]], gs[p], sem_g)
            pltpu.async_copy(ab_hbm.at[idx_d[p]], gd[p], sem_g)

        def wait_gathers(p):
            pltpu.make_async_copy(
                ab_hbm.at[pl.ds(0, BLK_B)], gs[p], sem_g).wait()
            pltpu.make_async_copy(
                ab_hbm.at[pl.ds(0, BLK_B)], gd[p], sem_g).wait()

        def wait_write(p):
            pltpu.make_async_copy(
                s_hbm.at[pl.ds(0, BLK_B)], sv[p], sem_w).wait()

        def phase(b, p, q):
            @pl.when(b + 1 < NBLK_B)
            def _():
                wait_idx(q)

                @pl.when(b >= 1)
                def _():
                    wait_write(q)

                issue_gathers(q)

            wait_gathers(p)
            sv[p][...] = gs[p][:, :32] + gd[p][:, 32:64]
            base = base0 + b * BLK_B
            pltpu.async_copy(sv[p], s_hbm.at[pl.ds(base, BLK_B)], sem_w)

            @pl.when(b + 2 < NBLK_B)
            def _():
                issue_idx(b + 2, p)

        issue_idx(0, 0)
        issue_idx(1, 1)
        wait_idx(0)
        issue_gathers(0)

        @pl.loop(0, NBLK_B, step=2)
        def _(t):
            phase(t, 0, 1)

            @pl.when(t + 1 < NBLK_B)
            def _():
                phase(t + 1, 1, 0)

        wait_write(0)
        wait_write(1)

    return sc_kernel(ab_tab, src, dst)


def _node_update_body(x_ref, ux_ref, agg_ref, gwu_ref, gwa_ref, gb_ref,
                      gamma_ref, beta_ref, o_ref):
    ux = ux_ref[...]
    agg = agg_ref[:N_NODES, :] + agg_ref[N_NODES:, :]
    logit = jnp.sum(ux * gwu_ref[...] + agg * gwa_ref[...],
                    axis=1, keepdims=True) + gb_ref[0:1, 0:1]
    gate = jax.nn.sigmoid(logit)
    h = gate * ux + (1.0 - gate) * agg
    mean = jnp.mean(h, axis=0, keepdims=True)
    var = jnp.mean((h - mean) ** 2, axis=0, keepdims=True)
    h_norm = (h - mean) / jnp.sqrt(var + 1e-5) * gamma_ref[...] + beta_ref[...]
    o_ref[...] = x_ref[...] + jax.nn.relu(h_norm)


def _node_update(x, ux, agg_partials, gwu, gwa, gb, gamma, beta):
    return pl.pallas_call(
        _node_update_body,
        out_shape=jax.ShapeDtypeStruct((N_NODES, D), jnp.float32),
    )(x, ux, agg_partials, gwu, gwa, gb, gamma, beta)


def _edge_mlp_body(ea_ref, s_ref, w3t_ref, b1_ref, w2t_ref, b2_ref, o_ref):
    c = jnp.dot(ea_ref[...], w3t_ref[...], preferred_element_type=jnp.float32)
    z = jax.nn.relu(s_ref[...] + c + b1_ref[...])
    o_ref[...] = jnp.dot(z, w2t_ref[...],
                         preferred_element_type=jnp.float32) + b2_ref[...]


def _edge_mlp(edge_attr, s, w3t, b1, w2t, b2):
    return pl.pallas_call(
        _edge_mlp_body,
        grid=(N_EDGES // E_BLK,),
        in_specs=[
            pl.BlockSpec((E_BLK, ED), lambda i: (i, 0)),
            pl.BlockSpec((E_BLK, 32), lambda i: (i, 0)),
            pl.BlockSpec((ED, 32), lambda i: (0, 0)),
            pl.BlockSpec((1, 32), lambda i: (0, 0)),
            pl.BlockSpec((32, ED), lambda i: (0, 0)),
            pl.BlockSpec((1, ED), lambda i: (0, 0)),
        ],
        out_specs=pl.BlockSpec((E_BLK, ED), lambda i: (i, 0)),
        out_shape=jax.ShapeDtypeStruct((N_EDGES, ED), jnp.float32),
    )(edge_attr, s, w3t, b1, w2t, b2)


def kernel(x, edge_index, edge_attr, U_w, U_b, V_w, V_b, E_w, E_b,
           gate_w, gate_b, eu1_w, eu1_b, eu2_w, eu2_b, bn_gamma, bn_beta):
    src = edge_index[0].astype(jnp.int32)
    dst = edge_index[1].astype(jnp.int32)

    w_cat = jnp.concatenate(
        [U_w.T, V_w.T, eu1_w[:, :D].T, eu1_w[:, D:2 * D].T,
         jnp.zeros((D, 64), jnp.float32)], axis=1)
    b_cat = jnp.concatenate(
        [U_b, V_b, jnp.zeros((D,), jnp.float32)])[None, :]

    ux, vx_tab, ab_tab = _node_matmuls(x, w_cat, b_cat)
    e = _edge_gate(edge_attr, E_w.T, E_b[None, :])

    zrows = jnp.zeros((ROWS_MAIN, D), jnp.float32)
    agg_partials = _sc_messages(vx_tab, e, src, dst, zrows)
    s_sum = _sc_edge_sums(ab_tab, src, dst)

    gwu = gate_w[:, :D]
    gwa = gate_w[:, D:]
    gb = jnp.broadcast_to(gate_b.reshape(1, 1), (1, D))
    x_new = _node_update(x, ux, agg_partials, gwu, gwa, gb,
                         bn_gamma[None, :], bn_beta[None, :])

    upd = _edge_mlp(edge_attr, s_sum, eu1_w[:, 2 * D:].T, eu1_b[None, :],
                    eu2_w.T, eu2_b[None, :])
    return x_new, upd
